# baseline (speedup 1.0000x reference)
import functools

import jax
import jax.numpy as jnp
from jax import lax
from jax.experimental import pallas as pl
from jax.experimental.pallas import tpu as pltpu
from jax.experimental.pallas import tpu_sc as plsc

_CHUNK = 128
_NBUF = 2


@functools.cache
def _make_gather(nf: int, nb: int, vocab: int, dim: int):
    info = plsc.get_sparse_core_info()
    nc, ns = info.num_cores, info.num_subcores
    nw = nc * ns
    n_total = nf * nb
    assert n_total % (nw * _CHUNK) == 0 and nb % _CHUNK == 0
    n_per_w = n_total // nw
    nchunk = n_per_w // _CHUNK
    cpj = nb // _CHUNK
    assert nchunk % _NBUF == 0

    mesh = plsc.VectorSubcoreMesh(core_axis_name="c", subcore_axis_name="s")

    @functools.partial(
        pl.kernel,
        mesh=mesh,
        out_type=jax.ShapeDtypeStruct((nf, dim, nb), jnp.float32),
        scratch_types=[
            pltpu.VMEM((n_per_w,), jnp.int32),
            pltpu.VMEM((_NBUF, _CHUNK, dim), jnp.float32),
            pltpu.VMEM((_NBUF, dim, _CHUNK), jnp.float32),
            pltpu.SemaphoreType.DMA((_NBUF,)),
            pltpu.SemaphoreType.DMA((_NBUF,)),
        ],
        compiler_params=pltpu.CompilerParams(
            use_tc_tiling_on_sc=False, needs_layout_passes=False
        ),
    )
    def gather_kernel(idx_hbm, table_hbm, out_hbm, idx_v, rows_v, tbuf_v,
                      gsem, osem):
        wid = lax.axis_index("s") * nc + lax.axis_index("c")
        base = wid * n_per_w
        pltpu.sync_copy(idx_hbm.at[pl.ds(base, n_per_w)], idx_v)
        iota = lax.iota(jnp.int32, 16)

        def start_gather(l, b):
            pltpu.async_copy(
                table_hbm.at[idx_v.at[pl.ds(l * _CHUNK, _CHUNK)]],
                rows_v.at[b],
                gsem.at[b],
            )

        def wait_gather(b):
            pltpu.make_async_copy(
                table_hbm.at[idx_v.at[pl.ds(0, _CHUNK)]],
                rows_v.at[b],
                gsem.at[b],
            ).wait()

        def start_out(l, b):
            gc = wid * nchunk + l
            j = gc // cpj
            b0 = (gc % cpj) * _CHUNK
            pltpu.async_copy(
                tbuf_v.at[b],
                out_hbm.at[j, :, pl.ds(b0, _CHUNK)],
                osem.at[b],
            )

        def wait_out(b):
            pltpu.make_async_copy(
                tbuf_v.at[b],
                out_hbm.at[0, :, pl.ds(0, _CHUNK)],
                osem.at[b],
            ).wait()

        for b in range(_NBUF):
            start_gather(b, b)

        @pl.loop(0, nchunk, step=_NBUF)
        def _(outer):
            for b in range(_NBUF):
                l = outer + b
                wait_gather(b)

                @pl.when(l >= _NBUF)
                def _():
                    wait_out(b)

                for f in range(dim):
                    fvec = jnp.full((16,), f, jnp.int32)
                    for g in range(_CHUNK // 16):
                        v = plsc.load_gather(
                            rows_v.at[b], [g * 16 + iota, fvec]
                        )
                        tbuf_v[b, f, pl.ds(g * 16, 16)] = v

                start_out(l, b)
                nxt = l + _NBUF

                @pl.when(nxt < nchunk)
                def _():
                    start_gather(nxt, b)

        for b in range(_NBUF):
            wait_out(b)

    return gather_kernel


def kernel(ids, table):
    nb, nf = ids.shape
    vocab, dim = table.shape
    ids_flat = ids.T.reshape(nf * nb)
    tflat = lax.optimization_barrier(table.reshape(vocab * dim))
    table_rm = tflat.reshape(vocab, dim)
    out_t = _make_gather(nf, nb, vocab, dim)(ids_flat, table_rm)
    return out_t.transpose(2, 0, 1)


# speedup vs baseline: 1.0421x; 1.0421x over previous
import functools

import jax
import jax.numpy as jnp
from jax import lax
from jax.experimental import pallas as pl
from jax.experimental.pallas import tpu as pltpu
from jax.experimental.pallas import tpu_sc as plsc

_CHUNK = 128
_NBUF = 2


@functools.cache
def _make_tableprep(vocab: int, dim: int):
    cb = 2048
    grid = (vocab + cb - 1) // cb

    def body(in_ref, out_ref):
        x = in_ref[...]
        t = jnp.transpose(x)
        y = t.reshape(cb // 2, 2, dim)
        out_ref[...] = jnp.concatenate([y[:, 0, :], y[:, 1, :]], axis=-1)

    return pl.pallas_call(
        body,
        grid=(grid,),
        in_specs=[pl.BlockSpec((dim, cb), lambda i: (0, i))],
        out_specs=pl.BlockSpec((cb // 2, 2 * dim), lambda i: (i, 0)),
        out_shape=jax.ShapeDtypeStruct((vocab // 2, 2 * dim), jnp.float32),
    )


@functools.cache
def _make_gather(nf: int, nb: int, vocab: int, dim: int):
    info = plsc.get_sparse_core_info()
    nc, ns = info.num_cores, info.num_subcores
    nw = nc * ns
    n_total = nf * nb
    assert n_total % (nw * _CHUNK) == 0 and nb % _CHUNK == 0
    n_per_w = n_total // nw
    nchunk = n_per_w // _CHUNK
    cpj = nb // _CHUNK
    assert nchunk % _NBUF == 0

    mesh = plsc.VectorSubcoreMesh(core_axis_name="c", subcore_axis_name="s")

    @functools.partial(
        pl.kernel,
        mesh=mesh,
        out_type=jax.ShapeDtypeStruct((nf, dim, nb), jnp.float32),
        scratch_types=[
            pltpu.VMEM((n_per_w,), jnp.int32),
            pltpu.VMEM((_NBUF, _CHUNK, dim), jnp.float32),
            pltpu.VMEM((_NBUF, dim, _CHUNK), jnp.float32),
            pltpu.SemaphoreType.DMA((_NBUF,)),
            pltpu.SemaphoreType.DMA((_NBUF,)),
        ],
        compiler_params=pltpu.CompilerParams(
            use_tc_tiling_on_sc=False, needs_layout_passes=False
        ),
    )
    def gather_kernel(idx_hbm, table_hbm, out_hbm, idx_v, rows_v, tbuf_v,
                      gsem, osem):
        wid = lax.axis_index("s") * nc + lax.axis_index("c")
        base = wid * n_per_w
        pltpu.sync_copy(idx_hbm.at[pl.ds(base, n_per_w)], idx_v)
        iota = lax.iota(jnp.int32, 16)

        def start_gather(l, b):
            pltpu.async_copy(
                table_hbm.at[idx_v.at[pl.ds(l * _CHUNK, _CHUNK)]],
                rows_v.at[b],
                gsem.at[b],
            )

        def wait_gather(b):
            pltpu.make_async_copy(
                table_hbm.at[idx_v.at[pl.ds(0, _CHUNK)]],
                rows_v.at[b],
                gsem.at[b],
            ).wait()

        def start_out(l, b):
            gc = wid * nchunk + l
            j = gc // cpj
            b0 = (gc % cpj) * _CHUNK
            pltpu.async_copy(
                tbuf_v.at[b],
                out_hbm.at[j, :, pl.ds(b0, _CHUNK)],
                osem.at[b],
            )

        def wait_out(b):
            pltpu.make_async_copy(
                tbuf_v.at[b],
                out_hbm.at[0, :, pl.ds(0, _CHUNK)],
                osem.at[b],
            ).wait()

        for b in range(_NBUF):
            start_gather(b, b)

        @pl.loop(0, nchunk, step=_NBUF)
        def _(outer):
            for b in range(_NBUF):
                l = outer + b
                wait_gather(b)

                @pl.when(l >= _NBUF)
                def _():
                    wait_out(b)

                for f in range(dim):
                    fvec = jnp.full((16,), f, jnp.int32)
                    for g in range(_CHUNK // 16):
                        v = plsc.load_gather(
                            rows_v.at[b], [g * 16 + iota, fvec]
                        )
                        tbuf_v[b, f, pl.ds(g * 16, 16)] = v

                start_out(l, b)
                nxt = l + _NBUF

                @pl.when(nxt < nchunk)
                def _():
                    start_gather(nxt, b)

        for b in range(_NBUF):
            wait_out(b)

    return gather_kernel


def kernel(ids, table):
    nb, nf = ids.shape
    vocab, dim = table.shape
    ids_flat = ids.T.reshape(nf * nb)
    t2 = _make_tableprep(vocab, dim)(table.T)
    table_rm = t2.reshape(vocab * dim).reshape(vocab, dim)
    out_t = _make_gather(nf, nb, vocab, dim)(ids_flat, table_rm)
    return out_t.transpose(2, 0, 1)


# speedup vs baseline: 1.7014x; 1.6327x over previous
import functools

import jax
import jax.numpy as jnp
from jax import lax
from jax.experimental import pallas as pl
from jax.experimental.pallas import tpu as pltpu
from jax.experimental.pallas import tpu_sc as plsc

_CHUNK = 512
_NBUF = 2


@functools.cache
def _make_tableprep(vocab: int, dim: int):
    cb = 2048
    grid = (vocab + cb - 1) // cb

    def body(in_ref, out_ref):
        x = in_ref[...]
        t = jnp.transpose(x)
        y = t.reshape(cb // 2, 2, dim)
        out_ref[...] = jnp.concatenate([y[:, 0, :], y[:, 1, :]], axis=-1)

    return pl.pallas_call(
        body,
        grid=(grid,),
        in_specs=[pl.BlockSpec((dim, cb), lambda i: (0, i))],
        out_specs=pl.BlockSpec((cb // 2, 2 * dim), lambda i: (i, 0)),
        out_shape=jax.ShapeDtypeStruct((vocab // 2, 2 * dim), jnp.float32),
    )


@functools.cache
def _make_gather(n_total: int, vocab: int, dim: int):
    info = plsc.get_sparse_core_info()
    nc, ns = info.num_cores, info.num_subcores
    nw = nc * ns
    assert n_total % (nw * _CHUNK * _NBUF) == 0
    n_per_w = n_total // nw
    nchunk = n_per_w // _CHUNK

    mesh = plsc.VectorSubcoreMesh(core_axis_name="c", subcore_axis_name="s")

    @functools.partial(
        pl.kernel,
        mesh=mesh,
        out_type=jax.ShapeDtypeStruct((n_total, dim), jnp.float32),
        scratch_types=[
            pltpu.VMEM((n_per_w,), jnp.int32),
            pltpu.VMEM((_NBUF, _CHUNK, dim), jnp.float32),
            pltpu.SemaphoreType.DMA((_NBUF,)),
            pltpu.SemaphoreType.DMA((_NBUF,)),
        ],
        compiler_params=pltpu.CompilerParams(use_tc_tiling_on_sc=False),
    )
    def gather_kernel(idx_hbm, table_hbm, out_hbm, idx_v, rows_v, gsem, osem):
        wid = lax.axis_index("s") * nc + lax.axis_index("c")
        base = wid * n_per_w
        pltpu.sync_copy(idx_hbm.at[pl.ds(base, n_per_w)], idx_v)

        def start_gather(g, b):
            pltpu.async_copy(
                table_hbm.at[idx_v.at[pl.ds(g * _CHUNK, _CHUNK)]],
                rows_v.at[b],
                gsem.at[b],
            )

        def wait_gather(b):
            pltpu.make_async_copy(
                table_hbm.at[idx_v.at[pl.ds(0, _CHUNK)]],
                rows_v.at[b],
                gsem.at[b],
            ).wait()

        def start_out(g, b):
            pltpu.async_copy(
                rows_v.at[b],
                out_hbm.at[pl.ds(base + g * _CHUNK, _CHUNK)],
                osem.at[b],
            )

        def wait_out(b):
            pltpu.make_async_copy(
                rows_v.at[b],
                out_hbm.at[pl.ds(base, _CHUNK)],
                osem.at[b],
            ).wait()

        for b in range(_NBUF):
            start_gather(b, b)

        @pl.loop(0, nchunk, step=_NBUF)
        def _(outer):
            for b in range(_NBUF):
                g = outer + b
                wait_gather(b)
                start_out(g, b)
                nxt = g + _NBUF

                @pl.when(nxt < nchunk)
                def _():
                    wait_out(b)
                    start_gather(nxt, b)

        for b in range(_NBUF):
            wait_out(b)

    return gather_kernel


def kernel(ids, table):
    nb, nf = ids.shape
    vocab, dim = table.shape
    n_total = nf * nb
    ids_flat = ids.T.reshape(n_total)
    t2 = _make_tableprep(vocab, dim)(table.T)
    table_rm = t2.reshape(vocab * dim).reshape(vocab, dim)
    rows = _make_gather(n_total, vocab, dim)(ids_flat, table_rm)
    return rows.reshape(nf, nb, dim).transpose(1, 0, 2)
